# baseline (speedup 1.0000x reference)
import functools

import jax
import jax.numpy as jnp
from jax import lax
from jax.experimental import pallas as pl
from jax.experimental.pallas import tpu as pltpu
from jax.experimental.pallas import tpu_sc as plsc

VOCAB = 1000000
EMBED = 64
B, T = 4096, 200

_info = plsc.get_sparse_core_info()
NC, NS, L = _info.num_cores, _info.num_subcores, _info.num_lanes
NW = NC * NS
ROWS_PER_W = B // NW
IDX_SPLIT = 2
IDX_MINOR = T // IDX_SPLIT
NB = 5
NO = 2


def _sc_body(
    x_hbm, table_hbm, pos_hbm, out_hbm, idx_all, pos_v, bufs, obufs, gsem, wsem
):
    wid = lax.axis_index("s") * NC + lax.axis_index("c")

    pltpu.sync_copy(x_hbm.at[wid], idx_all)
    pltpu.sync_copy(pos_hbm, pos_v)

    def fire_gather(c, slot):
        for j in range(IDX_SPLIT):
            pltpu.async_copy(
                table_hbm.at[idx_all.at[c, j]],
                bufs.at[slot, pl.ds(j * IDX_MINOR, IDX_MINOR)],
                gsem.at[slot],
            )

    for b in range(3):
        fire_gather(b, b)

    def loop_body(g, _):
        slot = lax.rem(g, NB)
        oslot = lax.rem(g, NO)
        pltpu.make_async_copy(
            out_hbm.at[wid, g], bufs.at[slot], gsem.at[slot]
        ).wait()

        nxt = g + 3

        @pl.when(nxt < ROWS_PER_W)
        def _():
            fire_gather(nxt, lax.rem(nxt, NB))

        @pl.when(g >= NO)
        def _():
            pltpu.make_async_copy(
                obufs.at[oslot], out_hbm.at[wid, g - NO], wsem.at[oslot]
            ).wait()

        @plsc.parallel_loop(0, T, unroll=8)
        def _add(t):
            for c in range(EMBED // L):
                sl = pl.ds(c * L, L)
                obufs[oslot, t, sl] = bufs[slot, t, sl] + pos_v[t, sl]

        pltpu.async_copy(obufs.at[oslot], out_hbm.at[wid, g], wsem.at[oslot])
        return 0

    lax.fori_loop(0, ROWS_PER_W, loop_body, 0)

    for k in range(NO):
        c = ROWS_PER_W - NO + k
        pltpu.make_async_copy(
            obufs.at[c % NO], out_hbm.at[wid, c], wsem.at[c % NO]
        ).wait()


@jax.jit
def kernel(x, token_table, pos_emb):
    x_r = x.astype(jnp.int32).reshape(NW, ROWS_PER_W, IDX_SPLIT, IDX_MINOR)
    pos_s = pos_emb[0, :T, :]

    mesh = plsc.VectorSubcoreMesh(core_axis_name="c", subcore_axis_name="s")
    sc_call = functools.partial(
        pl.kernel,
        mesh=mesh,
        out_type=jax.ShapeDtypeStruct((NW, ROWS_PER_W, T, EMBED), jnp.float32),
        scratch_types=[
            pltpu.VMEM((ROWS_PER_W, IDX_SPLIT, IDX_MINOR), jnp.int32),
            pltpu.VMEM((T, EMBED), jnp.float32),
            pltpu.VMEM((NB, T, EMBED), jnp.float32),
            pltpu.VMEM((NO, T, EMBED), jnp.float32),
            pltpu.SemaphoreType.DMA((NB,)),
            pltpu.SemaphoreType.DMA((NO,)),
        ],
        compiler_params=pltpu.CompilerParams(use_tc_tiling_on_sc=False),
    )(_sc_body)

    out = sc_call(x_r, token_table, pos_s)
    return out.reshape(B, T, EMBED)


# speedup vs baseline: 1.0033x; 1.0033x over previous
import functools

import jax
import jax.numpy as jnp
from jax import lax
from jax.experimental import pallas as pl
from jax.experimental.pallas import tpu as pltpu
from jax.experimental.pallas import tpu_sc as plsc

VOCAB = 1000000
EMBED = 64
B, T = 4096, 200

_info = plsc.get_sparse_core_info()
NC, NS, L = _info.num_cores, _info.num_subcores, _info.num_lanes
NW = NC * NS
ROWS_PER_W = B // NW
RPC = 2
NCHUNK = ROWS_PER_W // RPC
CR = RPC * T
IDX_SPLIT = 4
IDX_MINOR = CR // IDX_SPLIT
NB = 3


def _sc_body(x_hbm, table_hbm, pos_hbm, out_hbm, idx_all, pos_v, bufs, gsem, wsem):
    wid = lax.axis_index("s") * NC + lax.axis_index("c")

    pltpu.sync_copy(x_hbm.at[wid], idx_all)
    pltpu.sync_copy(pos_hbm, pos_v)

    def fire_gather(c, slot):
        for j in range(IDX_SPLIT):
            pltpu.async_copy(
                table_hbm.at[idx_all.at[c, j]],
                bufs.at[slot, pl.ds(j * IDX_MINOR, IDX_MINOR)],
                gsem.at[slot],
            )

    for b in range(NB - 1):
        fire_gather(b, b)

    def loop_body(g, _):
        slot = lax.rem(g, NB)
        pltpu.make_async_copy(
            out_hbm.at[wid, g], bufs.at[slot], gsem.at[slot]
        ).wait()

        @plsc.parallel_loop(0, T, unroll=8)
        def _add(t):
            for h in range(RPC):
                for c in range(EMBED // L):
                    sl = pl.ds(c * L, L)
                    r = h * T + t
                    bufs[slot, r, sl] = bufs[slot, r, sl] + pos_v[t, sl]

        pltpu.async_copy(bufs.at[slot], out_hbm.at[wid, g], wsem.at[slot])

        nxt = g + NB - 1

        @pl.when(nxt < NCHUNK)
        def _():
            slotn = lax.rem(nxt, NB)

            @pl.when(nxt >= NB)
            def _():
                pltpu.make_async_copy(
                    bufs.at[slotn], out_hbm.at[wid, nxt - NB], wsem.at[slotn]
                ).wait()

            fire_gather(nxt, slotn)

        return 0

    lax.fori_loop(0, NCHUNK, loop_body, 0)

    for k in range(NB):
        c = NCHUNK - NB + k
        pltpu.make_async_copy(
            bufs.at[c % NB], out_hbm.at[wid, c], wsem.at[c % NB]
        ).wait()


@jax.jit
def kernel(x, token_table, pos_emb):
    x_r = x.astype(jnp.int32).reshape(NW, NCHUNK, IDX_SPLIT, IDX_MINOR)
    pos_s = pos_emb[0, :T, :]

    mesh = plsc.VectorSubcoreMesh(core_axis_name="c", subcore_axis_name="s")
    sc_call = functools.partial(
        pl.kernel,
        mesh=mesh,
        out_type=jax.ShapeDtypeStruct((NW, NCHUNK, CR, EMBED), jnp.float32),
        scratch_types=[
            pltpu.VMEM((NCHUNK, IDX_SPLIT, IDX_MINOR), jnp.int32),
            pltpu.VMEM((T, EMBED), jnp.float32),
            pltpu.VMEM((NB, CR, EMBED), jnp.float32),
            pltpu.SemaphoreType.DMA((NB,)),
            pltpu.SemaphoreType.DMA((NB,)),
        ],
        compiler_params=pltpu.CompilerParams(use_tc_tiling_on_sc=False),
    )(_sc_body)

    out = sc_call(x_r, token_table, pos_s)
    return out.reshape(B, T, EMBED)
